# baseline (speedup 1.0000x reference)
import functools

import jax
import jax.numpy as jnp
from jax import lax
from jax.experimental import pallas as pl
from jax.experimental.pallas import tpu as pltpu
from jax.experimental.pallas import tpu_sc as plsc

N = 8192
D = 1024
NC = 2
NS = 16
NW = NC * NS
R = N // NW
C = 32
NCH = R // C
NBUF = 3
L = 16


def _make_lookup():
    mesh = plsc.VectorSubcoreMesh(core_axis_name="c", subcore_axis_name="s")
    scratch = [pltpu.VMEM((L,), jnp.int32),
               pltpu.VMEM((NCH, C), jnp.int32)]
    scratch += [pltpu.VMEM((C, D), jnp.float32) for _ in range(NBUF)]
    scratch += [pltpu.SemaphoreType.DMA for _ in range(1 + 2 * NBUF)]

    @functools.partial(
        pl.kernel,
        mesh=mesh,
        out_type=jax.ShapeDtypeStruct((N, D), jnp.float32),
        scratch_types=scratch,
    )
    def lookup_kernel(table_hbm, head_hbm, out_hbm, head_s, idx_v, *rest):
        bufs = rest[:NBUF]
        csem = rest[NBUF]
        gsem = rest[NBUF + 1:NBUF + 1 + NBUF]
        ssem = rest[NBUF + 1 + NBUF:]
        wid = lax.axis_index("s") * NC + lax.axis_index("c")
        row0 = wid * R

        pltpu.sync_copy(head_hbm, head_s)
        h = head_s[...][0]

        @pl.when(h == 0)
        def _fast():
            cp = pltpu.make_async_copy(
                table_hbm.at[pl.ds(row0, R)],
                out_hbm.at[pl.ds(row0, R)], csem)
            cp.start()
            cp.wait()

        @pl.when(h != 0)
        def _general():
            lanes = lax.iota(jnp.int32, L)
            for g in range(NCH):
                for sl in range(C // L):
                    base = row0 + g * C + sl * L
                    vec = lax.clamp(jnp.int32(0), lanes + (base - h),
                                    jnp.int32(N - 1))
                    idx_v[g, pl.ds(sl * L, L)] = vec

            def gather(g, b):
                return pltpu.make_async_copy(
                    table_hbm.at[idx_v.at[g]], bufs[b], gsem[b])

            def store(g, b):
                return pltpu.make_async_copy(
                    bufs[b], out_hbm.at[pl.ds(row0 + g * C, C)], ssem[b])

            gathers = [None] * NCH
            stores = [None] * NCH
            for g in range(min(NBUF, NCH)):
                gathers[g] = gather(g, g)
                gathers[g].start()
            for g in range(NCH):
                b = g % NBUF
                gathers[g].wait()
                stores[g] = store(g, b)
                stores[g].start()
                nxt = g + NBUF
                if nxt < NCH:
                    stores[g].wait()
                    gathers[nxt] = gather(nxt, b)
                    gathers[nxt].start()
            for g in range(max(NCH - NBUF, 0), NCH):
                stores[g].wait()

    return lookup_kernel


_lookup = _make_lookup()


@jax.jit
def kernel(seq_len, table):
    n, _ = table.shape
    head = jnp.full((L,), n - jnp.asarray(seq_len, jnp.int32), jnp.int32)
    return _lookup(table, head)


# speedup vs baseline: 23.3397x; 23.3397x over previous
import functools

import jax
import jax.numpy as jnp
from jax import lax
from jax.experimental import pallas as pl
from jax.experimental.pallas import tpu as pltpu
from jax.experimental.pallas import tpu_sc as plsc

N = 8192
D = 1024
NC = 2
NS = 16
NW = NC * NS
R = N // NW
C = 32
NCH = R // C
NBUF = 3
L = 16


def _make_lookup():
    mesh = plsc.VectorSubcoreMesh(core_axis_name="c", subcore_axis_name="s")
    scratch = [pltpu.VMEM((L,), jnp.int32),
               pltpu.VMEM((NCH, C), jnp.int32)]
    scratch += [pltpu.VMEM((C, D), jnp.float32) for _ in range(NBUF)]
    scratch += [pltpu.SemaphoreType.DMA for _ in range(2 * NBUF)]

    @functools.partial(
        pl.kernel,
        mesh=mesh,
        out_type=jax.ShapeDtypeStruct((N, D), jnp.float32),
        scratch_types=scratch,
    )
    def lookup_kernel(table_hbm, head_hbm, out_hbm, head_s, idx_v, *rest):
        bufs = rest[:NBUF]
        gsem = rest[NBUF:2 * NBUF]
        ssem = rest[2 * NBUF:]
        wid = lax.axis_index("s") * NC + lax.axis_index("c")
        row0 = wid * R

        pltpu.sync_copy(head_hbm, head_s)
        h = head_s[...][0]

        def run_ring(make_gather):
            def store(g, b):
                return pltpu.make_async_copy(
                    bufs[b], out_hbm.at[pl.ds(row0 + g * C, C)], ssem[b])

            gathers = [None] * NCH
            stores = [None] * NCH
            for g in range(min(NBUF, NCH)):
                gathers[g] = make_gather(g, g)
                gathers[g].start()
            for g in range(NCH):
                b = g % NBUF
                gathers[g].wait()
                stores[g] = store(g, b)
                stores[g].start()
                nxt = g + NBUF
                if nxt < NCH:
                    stores[g].wait()
                    gathers[nxt] = make_gather(nxt, b)
                    gathers[nxt].start()
            for g in range(max(NCH - NBUF, 0), NCH):
                stores[g].wait()

        @pl.when(h == 0)
        def _fast():
            def linear_gather(g, b):
                return pltpu.make_async_copy(
                    table_hbm.at[pl.ds(row0 + g * C, C)], bufs[b], gsem[b])
            run_ring(linear_gather)

        @pl.when(h != 0)
        def _general():
            lanes = lax.iota(jnp.int32, L)
            for g in range(NCH):
                for sl in range(C // L):
                    base = row0 + g * C + sl * L
                    vec = lax.clamp(jnp.int32(0), lanes + (base - h),
                                    jnp.int32(N - 1))
                    idx_v[g, pl.ds(sl * L, L)] = vec

            def indirect_gather(g, b):
                return pltpu.make_async_copy(
                    table_hbm.at[idx_v.at[g]], bufs[b], gsem[b])
            run_ring(indirect_gather)

    return lookup_kernel


_lookup = _make_lookup()


@jax.jit
def kernel(seq_len, table):
    n, _ = table.shape
    head = jnp.full((L,), n - jnp.asarray(seq_len, jnp.int32), jnp.int32)
    return _lookup(table, head)


# speedup vs baseline: 23.6289x; 1.0124x over previous
import functools

import jax
import jax.numpy as jnp
from jax import lax
from jax.experimental import pallas as pl
from jax.experimental.pallas import tpu as pltpu
from jax.experimental.pallas import tpu_sc as plsc

N = 8192
D = 1024
NC = 2
NS = 16
NW = NC * NS
R = N // NW
C = 16
NCH = R // C
NBUF = 7
L = 16


def _make_lookup():
    mesh = plsc.VectorSubcoreMesh(core_axis_name="c", subcore_axis_name="s")
    scratch = [pltpu.VMEM((L,), jnp.int32),
               pltpu.VMEM((NCH, C), jnp.int32)]
    scratch += [pltpu.VMEM((C, D), jnp.float32) for _ in range(NBUF)]
    scratch += [pltpu.SemaphoreType.DMA for _ in range(2 * NBUF)]

    @functools.partial(
        pl.kernel,
        mesh=mesh,
        out_type=jax.ShapeDtypeStruct((N, D), jnp.float32),
        scratch_types=scratch,
    )
    def lookup_kernel(table_hbm, head_hbm, out_hbm, head_s, idx_v, *rest):
        bufs = rest[:NBUF]
        gsem = rest[NBUF:2 * NBUF]
        ssem = rest[2 * NBUF:]
        wid = lax.axis_index("s") * NC + lax.axis_index("c")
        row0 = wid * R

        pltpu.sync_copy(head_hbm, head_s)
        h = head_s[...][0]

        def run_ring(make_gather):
            def store(g, b):
                return pltpu.make_async_copy(
                    bufs[b], out_hbm.at[pl.ds(row0 + g * C, C)], ssem[b])

            gathers = [None] * NCH
            stores = [None] * NCH
            for g in range(min(NBUF, NCH)):
                gathers[g] = make_gather(g, g)
                gathers[g].start()
            for g in range(NCH):
                b = g % NBUF
                gathers[g].wait()
                stores[g] = store(g, b)
                stores[g].start()
                nxt = g + NBUF
                if nxt < NCH:
                    stores[g].wait()
                    gathers[nxt] = make_gather(nxt, b)
                    gathers[nxt].start()
            for g in range(max(NCH - NBUF, 0), NCH):
                stores[g].wait()

        @pl.when(h == 0)
        def _fast():
            def linear_gather(g, b):
                return pltpu.make_async_copy(
                    table_hbm.at[pl.ds(row0 + g * C, C)], bufs[b], gsem[b])
            run_ring(linear_gather)

        @pl.when(h != 0)
        def _general():
            lanes = lax.iota(jnp.int32, L)
            for g in range(NCH):
                for sl in range(C // L):
                    base = row0 + g * C + sl * L
                    vec = lax.clamp(jnp.int32(0), lanes + (base - h),
                                    jnp.int32(N - 1))
                    idx_v[g, pl.ds(sl * L, L)] = vec

            def indirect_gather(g, b):
                return pltpu.make_async_copy(
                    table_hbm.at[idx_v.at[g]], bufs[b], gsem[b])
            run_ring(indirect_gather)

    return lookup_kernel


_lookup = _make_lookup()


@jax.jit
def kernel(seq_len, table):
    n, _ = table.shape
    head = jnp.full((L,), n - jnp.asarray(seq_len, jnp.int32), jnp.int32)
    return _lookup(table, head)
